# baseline (speedup 1.0000x reference)
import functools

import jax
import jax.numpy as jnp
from jax.experimental import pallas as pl
from jax.experimental.pallas import tpu as pltpu

_BUCKETS = 64
_DIM = 1024


def _fused_body(q_ref, k_ref, v_ref, w_ref, nk_ref, nv_ref, o_ref,
                kvk_ref, kvv_ref, bkr_ref, bvr_ref,
                ti_vmem, wt_vmem, ti_smem, wt_smem, sem0, sem1, *,
                h, hh, t, dh, buckets, bsz):
    neg = -jnp.finfo(jnp.float32).max
    pid = pl.program_id(0)
    is_rolled = (pid % h) >= hh
    scale = float(_DIM) ** -0.5

    shift = bsz - 1

    def roll_fwd(x):
        return jnp.concatenate([x[shift:], x[:shift]], axis=0)

    q = q_ref[0]
    k = k_ref[0]
    v = v_ref[0]
    q = jnp.where(is_rolled, roll_fwd(q), q)
    k = jnp.where(is_rolled, roll_fwd(k), k)
    v = jnp.where(is_rolled, roll_fwd(v), v)

    kb = k.reshape(buckets, bsz, dh)
    vb = v.reshape(buckets, bsz, dh)
    qb = q.reshape(buckets, bsz, dh) * scale

    bsums = jnp.sum(kb, axis=1)
    tri = (jax.lax.broadcasted_iota(jnp.int32, (buckets, buckets), 0)
           > jax.lax.broadcasted_iota(jnp.int32, (buckets, buckets), 1)
           ).astype(jnp.float32)
    excl = jnp.dot(tri, bsums, preferred_element_type=jnp.float32)
    firsts = kb[:, 0, :]
    pos = (jax.lax.broadcasted_iota(jnp.int32, (buckets, 1), 0) * bsz + 1
           ).astype(jnp.float32)
    x1 = (excl + firsts) / pos
    x = jnp.concatenate([x1, firsts], axis=1)

    r_raw = jnp.dot(x, w_ref[0], preferred_element_type=jnp.float32)
    r_act = jnp.where(r_raw >= 0, r_raw, 0.01 * r_raw)
    rows = jax.lax.broadcasted_iota(jnp.int32, (buckets, buckets + 1), 0)
    cols = jax.lax.broadcasted_iota(jnp.int32, (buckets, buckets + 1), 1)
    r_m = jnp.where(cols > rows, neg, r_act)
    r_m = r_m - jnp.max(r_m, axis=1, keepdims=True)
    r_e = jnp.exp(r_m)
    r_sm = r_e / jnp.sum(r_e, axis=1, keepdims=True)
    r_sm = jnp.where(cols <= rows - 1, r_sm, 0.0)

    mx_v = jnp.max(r_sm, axis=1, keepdims=True)
    top_v = jnp.min(jnp.where(r_sm == mx_v, cols, buckets + 1), axis=1,
                    keepdims=True)

    ti_vmem[...] = top_v
    wt_vmem[...] = mx_v
    cp_ti = pltpu.make_async_copy(ti_vmem, ti_smem, sem0)
    cp_wt = pltpu.make_async_copy(wt_vmem, wt_smem, sem1)
    cp_ti.start()
    cp_wt.start()

    kvk_ref[0:bsz, :] = jnp.broadcast_to(nk_ref[0], (bsz, dh))
    kvv_ref[0:bsz, :] = jnp.broadcast_to(nv_ref[0], (bsz, dh))
    kvk_ref[bsz:, :] = k
    kvv_ref[bsz:, :] = v
    cp_ti.wait()
    cp_wt.wait()

    for u in range(buckets):
        src = ti_smem[u, 0] * bsz
        w_u = wt_smem[u, 0]
        bkr_ref[pl.ds(u * bsz, bsz), :] = w_u * kvk_ref[pl.ds(src, bsz), :]
        bvr_ref[pl.ds(u * bsz, bsz), :] = w_u * kvv_ref[pl.ds(src, bsz), :]

    bkr = bkr_ref[...].reshape(buckets, bsz, dh)
    bvr = bvr_ref[...].reshape(buckets, bsz, dh)

    bdims = (((2,), (2,)), ((0,), (0,)))
    dots_r = jax.lax.dot_general(qb, bkr, bdims,
                                 preferred_element_type=jnp.float32)
    dots_s = jax.lax.dot_general(qb, kb, bdims,
                                 preferred_element_type=jnp.float32)

    ii = jax.lax.broadcasted_iota(jnp.int32, (bsz, bsz), 0)
    jj = jax.lax.broadcasted_iota(jnp.int32, (bsz, bsz), 1)
    ub = jax.lax.broadcasted_iota(jnp.int32, (buckets, 1, 1), 0)
    causal_f = jnp.where(jj <= ii, 0.0, neg)
    is_last = (ub == buckets - 1) & is_rolled

    row_mask_f = jnp.where((ii > 0) | (jj >= bsz), neg, 0.0)[:, 0:1]
    mask_r = jnp.where(is_last, row_mask_f[None], 0.0)
    spc_f = jnp.where(((ii > 0) & (jj == 0)) | ((ii == 0) & (jj > 0)),
                      neg, 0.0)
    mask_s = causal_f[None] + jnp.where(is_last, spc_f[None], 0.0)

    e_r = jnp.exp(dots_r + mask_r)
    e_s = jnp.exp(dots_s + mask_s)
    denom = jnp.sum(e_r, axis=2, keepdims=True) + \
        jnp.sum(e_s, axis=2, keepdims=True)
    inv = 1.0 / denom
    a_r = e_r * inv
    a_s = e_s * inv
    odims = (((2,), (1,)), ((0,), (0,)))
    ob = jax.lax.dot_general(a_r, bvr, odims,
                             preferred_element_type=jnp.float32)
    ob = ob + jax.lax.dot_general(a_s, vb, odims,
                                  preferred_element_type=jnp.float32)

    o = ob.reshape(t, dh)
    o_roll = jnp.concatenate([o[t - shift:], o[:t - shift]], axis=0)
    o_ref[0] = jnp.where(is_rolled, o_roll, o)


def kernel(q, k, v, null_keys, null_values, sort_linear):
    b, h, t, dh = q.shape
    bh = b * h
    buckets = _BUCKETS
    bsz = t // buckets
    hh = h // 2

    qf = q.reshape(bh, t, dh)
    kf = k.reshape(bh, t, dh)
    vf = v.reshape(bh, t, dh)
    w = sort_linear.reshape(h, 2 * dh, buckets + 1)
    nk = null_keys.reshape(h, 1, dh)
    nv = null_values.reshape(h, 1, dh)

    body = functools.partial(_fused_body, h=h, hh=hh, t=t, dh=dh,
                             buckets=buckets, bsz=bsz)
    out = pl.pallas_call(
        body,
        grid=(bh,),
        in_specs=[
            pl.BlockSpec((1, t, dh), lambda i: (i, 0, 0)),
            pl.BlockSpec((1, t, dh), lambda i: (i, 0, 0)),
            pl.BlockSpec((1, t, dh), lambda i: (i, 0, 0)),
            pl.BlockSpec((1, 2 * dh, buckets + 1), lambda i, h=h: (i % h, 0, 0)),
            pl.BlockSpec((1, 1, dh), lambda i, h=h: (i % h, 0, 0)),
            pl.BlockSpec((1, 1, dh), lambda i, h=h: (i % h, 0, 0)),
        ],
        out_specs=pl.BlockSpec((1, t, dh), lambda i: (i, 0, 0)),
        out_shape=jax.ShapeDtypeStruct((bh, t, dh), jnp.float32),
        scratch_shapes=[
            pltpu.VMEM((bsz + t, dh), jnp.float32),
            pltpu.VMEM((bsz + t, dh), jnp.float32),
            pltpu.VMEM((t, dh), jnp.float32),
            pltpu.VMEM((t, dh), jnp.float32),
            pltpu.VMEM((buckets, 1), jnp.int32),
            pltpu.VMEM((buckets, 1), jnp.float32),
            pltpu.SMEM((buckets, 1), jnp.int32),
            pltpu.SMEM((buckets, 1), jnp.float32),
            pltpu.SemaphoreType.DMA,
            pltpu.SemaphoreType.DMA,
        ],
        compiler_params=pltpu.CompilerParams(
            dimension_semantics=("parallel",)),
    )(qf, kf, vf, w, nk, nv)
    return out.reshape(b, h, t, dh)
